# baseline (speedup 1.0000x reference)
import functools

import jax
import jax.numpy as jnp
from jax import lax
from jax.experimental import pallas as pl
from jax.experimental.pallas import tpu as pltpu
from jax.experimental.pallas import tpu_sc as plsc

_NUM_CORES = 2
_NUM_SUBCORES = 16
_NUM_WORKERS = _NUM_CORES * _NUM_SUBCORES
_MAX_CHUNK_ROWS = 32
_NBUF = 3


def _chunk_sizes(rows):
    sizes = []
    for s in (8, 16, 24):
        if sum(sizes) + s <= rows:
            sizes.append(s)
    while rows - sum(sizes) >= _MAX_CHUNK_ROWS:
        sizes.append(_MAX_CHUNK_ROWS)
    if rows - sum(sizes) > 0:
        sizes.append(rows - sum(sizes))
    return tuple(sizes)


@functools.partial(jax.jit, static_argnums=(1, 2))
def _position_embedding(table, seq_len, dim):
    rows_per_worker = seq_len // _NUM_WORKERS
    sizes = _chunk_sizes(rows_per_worker)
    offs = [0]
    for s in sizes:
        offs.append(offs[-1] + s)
    n_chunks = len(sizes)
    mesh = plsc.VectorSubcoreMesh(
        core_axis_name="c", subcore_axis_name="s", num_cores=_NUM_CORES
    )

    @functools.partial(
        pl.kernel,
        out_type=jax.ShapeDtypeStruct((seq_len, dim), table.dtype),
        mesh=mesh,
        scratch_types=[
            pltpu.VMEM((_NBUF, _MAX_CHUNK_ROWS, dim), table.dtype),
            pltpu.SemaphoreType.DMA((_NBUF,)),
            pltpu.SemaphoreType.DMA((_NBUF,)),
        ],
    )
    def copy_kernel(table_hbm, out_hbm, buf, in_sems, out_sems):
        wid = lax.axis_index("s") * _NUM_CORES + lax.axis_index("c")
        base = wid * rows_per_worker

        def fire_in(c):
            b = c % _NBUF
            return pltpu.async_copy(
                table_hbm.at[pl.ds(base + offs[c], sizes[c])],
                buf.at[b, pl.ds(0, sizes[c])],
                in_sems.at[b],
            )

        def fire_out(c):
            b = c % _NBUF
            return pltpu.async_copy(
                buf.at[b, pl.ds(0, sizes[c])],
                out_hbm.at[pl.ds(base + offs[c], sizes[c])],
                out_sems.at[b],
            )

        in_dma, out_dma = {}, {}
        for c in range(min(_NBUF, n_chunks)):
            in_dma[c] = fire_in(c)
        fired = min(_NBUF, n_chunks)
        unwaited = []
        for c in range(n_chunks):
            in_dma[c].wait()
            out_dma[c] = fire_out(c)
            unwaited.append(c)
            if fired < n_chunks:
                out_dma[fired - _NBUF].wait()
                unwaited.remove(fired - _NBUF)
                in_dma[fired] = fire_in(fired)
                fired += 1
        for c in unwaited:
            out_dma[c].wait()

    return copy_kernel(table)


def kernel(inputs, table):
    seq_len = inputs.shape[-1]
    return _position_embedding(table, seq_len, table.shape[1])


# speedup vs baseline: 1.0378x; 1.0378x over previous
import functools

import jax
import jax.numpy as jnp
from jax import lax
from jax.experimental import pallas as pl
from jax.experimental.pallas import tpu as pltpu
from jax.experimental.pallas import tpu_sc as plsc

_NUM_CORES = 2
_NUM_SUBCORES = 16
_NUM_WORKERS = _NUM_CORES * _NUM_SUBCORES
_MAX_CHUNK_ROWS = 40
_NBUF = 3


def _chunk_sizes(rows):
    if rows == 128:
        return (24, 40, 32, 32)
    sizes = []
    while rows - sum(sizes) >= _MAX_CHUNK_ROWS:
        sizes.append(_MAX_CHUNK_ROWS)
    if rows - sum(sizes) > 0:
        sizes.append(rows - sum(sizes))
    return tuple(sizes)


@functools.partial(jax.jit, static_argnums=(1, 2))
def _position_embedding(table, seq_len, dim):
    rows_per_worker = seq_len // _NUM_WORKERS
    sizes = _chunk_sizes(rows_per_worker)
    offs = [0]
    for s in sizes:
        offs.append(offs[-1] + s)
    n_chunks = len(sizes)
    mesh = plsc.VectorSubcoreMesh(
        core_axis_name="c", subcore_axis_name="s", num_cores=_NUM_CORES
    )

    @functools.partial(
        pl.kernel,
        out_type=jax.ShapeDtypeStruct((seq_len, dim), table.dtype),
        mesh=mesh,
        scratch_types=[
            pltpu.VMEM((_NBUF, _MAX_CHUNK_ROWS, dim), table.dtype),
            pltpu.SemaphoreType.DMA((_NBUF,)),
            pltpu.SemaphoreType.DMA((_NBUF,)),
        ],
    )
    def copy_kernel(table_hbm, out_hbm, buf, in_sems, out_sems):
        wid = lax.axis_index("s") * _NUM_CORES + lax.axis_index("c")
        base = wid * rows_per_worker

        def fire_in(c):
            b = c % _NBUF
            return pltpu.async_copy(
                table_hbm.at[pl.ds(base + offs[c], sizes[c])],
                buf.at[b, pl.ds(0, sizes[c])],
                in_sems.at[b],
            )

        def fire_out(c):
            b = c % _NBUF
            return pltpu.async_copy(
                buf.at[b, pl.ds(0, sizes[c])],
                out_hbm.at[pl.ds(base + offs[c], sizes[c])],
                out_sems.at[b],
            )

        in_dma, out_dma = {}, {}
        for c in range(min(_NBUF, n_chunks)):
            in_dma[c] = fire_in(c)
        fired = min(_NBUF, n_chunks)
        unwaited = []
        for c in range(n_chunks):
            in_dma[c].wait()
            out_dma[c] = fire_out(c)
            unwaited.append(c)
            if fired < n_chunks:
                out_dma[fired - _NBUF].wait()
                unwaited.remove(fired - _NBUF)
                in_dma[fired] = fire_in(fired)
                fired += 1
        for c in unwaited:
            out_dma[c].wait()

    return copy_kernel(table)


def kernel(inputs, table):
    seq_len = inputs.shape[-1]
    return _position_embedding(table, seq_len, table.shape[1])
